# baseline (speedup 1.0000x reference)
import jax
import jax.numpy as jnp
from jax import lax
from jax.experimental import pallas as pl
from jax.experimental.pallas import tpu as pltpu
from jax.experimental.pallas import tpu_sc as plsc


def _make_kernel(b, f, h, w):
    mesh = plsc.VectorSubcoreMesh(core_axis_name="c", subcore_axis_name="s")
    info = plsc.get_sparse_core_info()
    nc, ns = info.num_cores, info.num_subcores
    nw = nc * ns
    rows_per_w = 2 * f // nw

    def body(xet_hbm, yet_hbm, o_hbm, xet_v, yet_v, slab_v, sem):
        wid = lax.axis_index("s") * nc + lax.axis_index("c")
        c0 = rows_per_w * wid
        pltpu.sync_copy(xet_hbm, xet_v)
        pltpu.sync_copy(yet_hbm, yet_v)

        def build_x():
            def cl_body(cl, _):
                c = c0 + cl
                v0 = xet_v[c, pl.ds(0, 16)]
                v1 = xet_v[c, pl.ds(16, 16)]

                def i_body(i, _):
                    slab_v[cl, i, pl.ds(0, 16)] = v0
                    slab_v[cl, i, pl.ds(16, 16)] = v1
                    return 0

                return lax.fori_loop(0, h, i_body, 0) * 0

            lax.fori_loop(0, rows_per_w, cl_body, 0)

        def build_y():
            def cl_body(cl, _):
                cevec = jnp.full((16,), c0 - f + cl, jnp.int32)

                def i_body(i, _):
                    sv = plsc.load_gather(
                        yet_v, [cevec, jnp.full((16,), i, jnp.int32)])
                    slab_v[cl, i, pl.ds(0, 16)] = sv
                    slab_v[cl, i, pl.ds(16, 16)] = sv
                    return 0

                return lax.fori_loop(0, h, i_body, 0) * 0

            lax.fori_loop(0, rows_per_w, cl_body, 0)

        lax.cond(wid < f // rows_per_w, build_x, build_y)

        copies = [
            pltpu.async_copy(slab_v, o_hbm.at[bb, pl.ds(c0, rows_per_w)], sem)
            for bb in range(b)
        ]
        for cp in copies:
            cp.wait()

    return pl.kernel(
        body,
        mesh=mesh,
        out_type=jax.ShapeDtypeStruct((b, 2 * f, h, w), jnp.float32),
        scratch_types=[
            pltpu.VMEM((f, w), jnp.float32),
            pltpu.VMEM((f, h), jnp.float32),
            pltpu.VMEM((rows_per_w, h, w), jnp.float32),
            pltpu.SemaphoreType.DMA,
        ],
        compiler_params=pltpu.CompilerParams(
            use_tc_tiling_on_sc=False, needs_layout_passes=False),
    )


def kernel(x, xenc, yenc):
    b = x.shape[0]
    h, w = x.shape[-2], x.shape[-1]
    f = xenc.shape[1]
    xet = jnp.transpose(xenc[:w])
    yet = jnp.transpose(yenc[:h])
    return _make_kernel(b, f, h, w)(xet, yet)


# speedup vs baseline: 3.5905x; 3.5905x over previous
import jax
import jax.numpy as jnp
from jax import lax
from jax.experimental import pallas as pl
from jax.experimental.pallas import tpu as pltpu


def _make_body(b, f, h, w):
    hw = h * w

    def body(xe_ref, ye_ref, o_ref, scratch_ref, sem):
        k = lax.broadcasted_iota(jnp.int32, (w, hw), 1)
        r = lax.broadcasted_iota(jnp.int32, (w, hw), 0)
        sel_x = (k % w == r).astype(jnp.float32)
        sel_y = (k // w == r).astype(jnp.float32)
        dn = (((0,), (0,)), ((), ()))
        scratch_ref[:f] = lax.dot_general(
            xe_ref[...], sel_x, dn, preferred_element_type=jnp.float32)
        scratch_ref[f:] = lax.dot_general(
            ye_ref[...], sel_y, dn, preferred_element_type=jnp.float32)
        for i in range(b):
            pltpu.make_async_copy(scratch_ref, o_ref.at[i], sem).start()
        for i in range(b):
            pltpu.make_async_copy(scratch_ref, o_ref.at[i], sem).wait()

    return body


def kernel(x, xenc, yenc):
    b = x.shape[0]
    h, w = x.shape[-2], x.shape[-1]
    f = xenc.shape[1]
    out = pl.pallas_call(
        _make_body(b, f, h, w),
        in_specs=[
            pl.BlockSpec(memory_space=pltpu.MemorySpace.VMEM),
            pl.BlockSpec(memory_space=pltpu.MemorySpace.VMEM),
        ],
        out_specs=pl.BlockSpec(memory_space=pltpu.MemorySpace.HBM),
        out_shape=jax.ShapeDtypeStruct((b, 2 * f, h * w), jnp.float32),
        scratch_shapes=[
            pltpu.VMEM((2 * f, h * w), jnp.float32),
            pltpu.SemaphoreType.DMA,
        ],
    )(xenc[:w], yenc[:h])
    return out.reshape(b, 2 * f, h, w)
